# baseline (speedup 1.0000x reference)
import math

import jax
import jax.numpy as jnp
from jax.experimental import pallas as pl
from jax.experimental.pallas import tpu as pltpu

_C = 100000
_SMOOTHING = 0.1
_CONF = 1.0 - _SMOOTHING
_EPS = _SMOOTHING / (_C - 1)

_R = 64
_L = 128
_NT = -(-_C // _L)
_CPAD = _NT * _L
_TAIL_REM = _C - (_NT - 1) * _L
_NCH = 16
_SEC = 8192
_NSEC = -(-_C // _SEC)
_NEG_INF = float("-inf")


def _tree(vals, op):
    while len(vals) > 1:
        nxt = [op(vals[i], vals[i + 1]) for i in range(0, len(vals) - 1, 2)]
        if len(vals) % 2:
            nxt.append(vals[-1])
        vals = nxt
    return vals[0]


def _tile(pred_ref, t):
    return pred_ref[:, pl.ds(t * _L, _L)]


def _lane_iota():
    return jax.lax.broadcasted_iota(jnp.int32, (_R, _L), 1)


def _chains(n):
    per = -(-n // _NCH)
    for c in range(_NCH):
        lo = c * per
        hi = min(lo + per, n)
        if lo < hi:
            yield range(lo, hi)


def _masked(x, t, fill):
    if t == _NT - 1:
        return jnp.where(_lane_iota() < _TAIL_REM, x, fill)
    return x


def _sweep_max(pred_ref):
    accs = []
    for chain in _chains(_NT):
        acc = None
        for t in chain:
            x = _masked(_tile(pred_ref, t), t, _NEG_INF)
            acc = x if acc is None else jnp.maximum(acc, x)
        accs.append(acc)
    return _tree(accs, jnp.maximum)


def _sweep_stats(pred_ref, m):
    se_accs = []
    sp_accs = []
    for chain in _chains(_NT):
        se = None
        sp = None
        for t in chain:
            x = _tile(pred_ref, t)
            e = jnp.exp(_masked(x, t, _NEG_INF) - m)
            xs = _masked(x, t, 0.0)
            se = e if se is None else se + e
            sp = xs if sp is None else sp + xs
        se_accs.append(se)
        sp_accs.append(sp)
    return _tree(se_accs, jnp.add), _tree(sp_accs, jnp.add)


def _loss_kernel(tgt_s_ref, tgt_v_ref, pred_ref, out_ref, ts_ref):
    rb = pl.program_id(0)

    ts_ref[...] = jnp.zeros((_R, _L), jnp.float32)

    tgtv = jnp.broadcast_to(tgt_v_ref[0, 0, :].reshape(_R, 1), (_R, _L))
    li = _lane_iota()
    for sec in range(_NSEC):
        lo = sec * _SEC
        hi = min(lo + _SEC, _C)
        hit = None
        for i in range(_R):
            t = tgt_s_ref[0, 0, i]
            h = jnp.logical_and(t >= lo, t < hi)
            hit = h if hit is None else jnp.logical_or(hit, h)

        @pl.when(hit)
        def _tsel(lo=lo, hi=hi):
            accs = []
            t0 = lo // _L
            t1 = -(-hi // _L)
            for g0 in range(t0, t1, 8):
                acc = None
                for t in range(g0, min(g0 + 8, t1)):
                    col = li + t * _L
                    v = jnp.where(col == tgtv, _tile(pred_ref, t), 0.0)
                    acc = v if acc is None else acc + v
                accs.append(acc)
            ts_ref[...] = ts_ref[...] + _tree(accs, jnp.add)

    m = _sweep_max(pred_ref)
    se, sp = _sweep_stats(pred_ref, m)

    mx = jnp.max(m, axis=1, keepdims=True)
    sx = jnp.sum(se * jnp.exp(m - mx), axis=1, keepdims=True)
    spx = jnp.sum(sp, axis=1, keepdims=True)
    tsx = jnp.sum(ts_ref[...], axis=1, keepdims=True)
    lse = mx + jnp.log(sx)
    rowsum_logp = spx - _C * lse
    logp_t = tsx - lse
    contrib = -(_EPS * rowsum_logp + (_CONF - _EPS) * logp_t)
    val = jnp.sum(contrib)

    @pl.when(rb == 0)
    def _():
        out_ref[0, 0] = val

    @pl.when(rb > 0)
    def _():
        out_ref[0, 0] = out_ref[0, 0] + val


@jax.jit
def kernel(pred, target):
    B = pred.shape[0]
    nb = B // _R
    tgt3 = target.astype(jnp.int32).reshape(nb, 1, _R)

    acc = pl.pallas_call(
        _loss_kernel,
        grid=(nb,),
        in_specs=[
            pl.BlockSpec((1, 1, _R), lambda rb: (rb, 0, 0),
                         memory_space=pltpu.SMEM),
            pl.BlockSpec((1, 1, _R), lambda rb: (rb, 0, 0)),
            pl.BlockSpec((_R, _CPAD), lambda rb: (rb, 0)),
        ],
        out_specs=pl.BlockSpec(
            (1, 1), lambda rb: (0, 0), memory_space=pltpu.SMEM),
        out_shape=jax.ShapeDtypeStruct((1, 1), jnp.float32),
        scratch_shapes=[
            pltpu.VMEM((_R, _L), jnp.float32),
        ],
    )(tgt3, tgt3, pred)

    k0 = (_C - 1) * _EPS * math.log(_EPS) + _CONF * math.log(_CONF)
    return (acc[0, 0] + B * k0) / (B * _C)
